# Initial kernel scaffold; baseline (speedup 1.0000x reference)
import jax
import jax.numpy as jnp
from jax.experimental import pallas as pl


def kernel(x, t, e_t, e_xct, unique_mask, params):
    raise NotImplementedError("write your pallas kernel here")


# speedup vs baseline: 9.7530x; 9.7530x over previous
import functools

import jax
import jax.numpy as jnp
from jax import lax
from jax.experimental import pallas as pl
from jax.experimental.pallas import tpu as pltpu
from jax.experimental.pallas import tpu_sc as plsc

_NT = 4096
_D = 128
_CH = 128
_NW = 32
_F32 = jnp.float32


def _seg_sum_body(chunk_counts, *refs):
    n_sets = len(chunk_counts)
    i = 0
    zeros = refs[i]; i += 1
    tables, srcs, dsts = [], [], []
    for _ in range(n_sets):
        tables.append(refs[i]); srcs.append(refs[i + 1]); dsts.append(refs[i + 2])
        i += 3
    outs = refs[i:i + n_sets]; i += n_sets
    scr = refs[i:]
    idx_v = [(scr[2 * k], scr[2 * k + 1]) for k in range(n_sets)]
    rows_v = scr[2 * n_sets]
    shs = scr[2 * n_sets + 1: 2 * n_sets + 1 + n_sets]
    sem = scr[-1]

    c = lax.axis_index("c")
    s = lax.axis_index("s")
    w = c * 16 + s
    rpt = _NT // 16

    for k in range(n_sets):
        pltpu.sync_copy(zeros.at[pl.ds(s * rpt, rpt)], shs[k].at[pl.ds(s * rpt, rpt)])
    plsc.subcore_barrier()

    for k in range(n_sets):
        nper = chunk_counts[k] // _NW
        src_v, dst_v = idx_v[k]
        pltpu.sync_copy(srcs[k].at[pl.ds(w * nper, nper)], src_v)
        pltpu.sync_copy(dsts[k].at[pl.ds(w * nper, nper)], dst_v)

        def loop_body(j, carry, k=k, src_v=src_v, dst_v=dst_v):
            pltpu.async_copy(tables[k].at[src_v.at[j]], rows_v, sem).wait()
            pltpu.sync_copy(rows_v, shs[k].at[dst_v.at[j]], add=True)
            return carry

        lax.fori_loop(0, nper, loop_body, 0)

    plsc.subcore_barrier()
    for k in range(n_sets):
        pltpu.sync_copy(shs[k].at[pl.ds(s * rpt, rpt)],
                        outs[k].at[c, pl.ds(s * rpt, rpt)])


@functools.cache
def _make_seg_sum(chunk_counts):
    mesh = plsc.VectorSubcoreMesh(core_axis_name="c", subcore_axis_name="s")
    n_sets = len(chunk_counts)
    out_type = tuple(jax.ShapeDtypeStruct((2, _NT, _D), _F32) for _ in range(n_sets))
    scratch = []
    for nc in chunk_counts:
        nper = nc // _NW
        scratch += [pltpu.VMEM((nper, _CH), jnp.int32),
                    pltpu.VMEM((nper, _CH), jnp.int32)]
    scratch.append(pltpu.VMEM((_CH, _D), _F32))
    scratch += [pltpu.VMEM_SHARED((_NT, _D), _F32) for _ in range(n_sets)]
    scratch.append(pltpu.SemaphoreType.DMA)
    body = functools.partial(_seg_sum_body, chunk_counts)
    return pl.kernel(body, out_type=out_type, mesh=mesh,
                     scratch_types=tuple(scratch),
                     name=f"seg_sum_{n_sets}")


def _dot(a, b):
    return jnp.dot(a, b, preferred_element_type=_F32)


def _proj_body(x_ref, t_ref, wx_ref, bx_ref, wt_ref, bt_ref, xs_ref, tf_ref):
    xs_ref[...] = jnp.maximum(_dot(x_ref[...], wx_ref[...]) + bx_ref[...], 0.0)
    tf_ref[...] = jnp.maximum(_dot(t_ref[...], wt_ref[...]) + bt_ref[...], 0.0)


def _gin_pair(tf, at0, at1, ax0, ax1, w1t, b1t, w2t, b2t, w1x, b1x, w2x, b2x):
    ht = tf + at0 + at1
    hx = tf + ax0 + ax1
    tt = _dot(jnp.maximum(_dot(ht, w1t) + b1t, 0.0), w2t) + b2t
    xct = _dot(jnp.maximum(_dot(hx, w1x) + b1x, 0.0), w2x) + b2x
    return jnp.maximum(tf + tt + xct, 0.0)


def _post(tf2, wo, bo, g, bl):
    h = jnp.maximum(_dot(tf2, wo) + bo, 0.0)
    mu = jnp.mean(h, axis=-1, keepdims=True)
    var = jnp.mean((h - mu) ** 2, axis=-1, keepdims=True)
    hn = (h - mu) / jnp.sqrt(var + 1e-5) * g + bl
    return tf2 + hn


def _blk_body(tf_ref, at_ref, ax_ref, w1t, b1t, w2t, b2t, w1x, b1x, w2x, b2x,
              wo, bo, g, bl, out_ref):
    tf2 = _gin_pair(tf_ref[...], at_ref[0], at_ref[1], ax_ref[0], ax_ref[1],
                    w1t[...], b1t[...], w2t[...], b2t[...],
                    w1x[...], b1x[...], w2x[...], b2x[...])
    out_ref[...] = _post(tf2, wo[...], bo[...], g[...], bl[...])


def _final_body(tf_ref, mask_ref, at_ref, ax_ref, w1t, b1t, w2t, b2t,
                w1x, b1x, w2x, b2x, wo, bo, g, bl, wsc, bsc, out_ref):
    tf2 = _gin_pair(tf_ref[...], at_ref[0], at_ref[1], ax_ref[0], ax_ref[1],
                    w1t[...], b1t[...], w2t[...], b2t[...],
                    w1x[...], b1x[...], w2x[...], b2x[...])
    tff = _post(tf2, wo[...], bo[...], g[...], bl[...])
    s = _dot(tff, wsc[...]) + bsc[...]
    out_ref[...] = jnp.where(mask_ref[...] != 0, s, -jnp.inf)


_ROWS_BLK = 1024


def _row_spec(shape):
    nd = len(shape)
    blk = (_ROWS_BLK,) + shape[1:]
    if nd == 2:
        return pl.BlockSpec(blk, lambda i: (i, 0))
    return pl.BlockSpec(blk, lambda i: (i, 0, 0))


def _full_spec(shape):
    nd = len(shape)
    return pl.BlockSpec(shape, (lambda i: (0, 0)) if nd == 2 else (lambda i: (0, 0, 0)))


def _part_spec(shape):
    return pl.BlockSpec((2, _ROWS_BLK, shape[2]), lambda i: (0, i, 0))


def _run_blocked(body, row_args, part_args, full_args, out_shape):
    grid = (_NT // _ROWS_BLK,)
    in_specs = ([_row_spec(a.shape) for a in row_args]
                + [_part_spec(a.shape) for a in part_args]
                + [_full_spec(a.shape) for a in full_args])
    return pl.pallas_call(
        body,
        grid=grid,
        in_specs=in_specs,
        out_specs=_row_spec(out_shape.shape),
        out_shape=out_shape,
    )(*row_args, *part_args, *full_args)


def kernel(x, t, e_t, e_xct, unique_mask, params):
    nb, nn_, dn = x.shape
    _, nr, dr = t.shape
    x4 = x.reshape(-1, dn)[:_NT]
    t2 = t.reshape(-1, dr)

    et = e_t.astype(jnp.int32).reshape(2, -1, _CH)
    ex = e_xct.astype(jnp.int32).reshape(2, -1, _CH)
    nct = et.shape[1]
    ncx = ex.shape[1]

    p = params
    bx = p['bx'].reshape(1, _D)
    bt = p['bt'].reshape(1, _D)

    xs, tf = pl.pallas_call(
        _proj_body,
        grid=(_NT // _ROWS_BLK,),
        in_specs=[_row_spec((_NT, _D)), _row_spec((_NT, _D)),
                  _full_spec((_D, _D)), _full_spec((1, _D)),
                  _full_spec((_D, _D)), _full_spec((1, _D))],
        out_specs=(_row_spec((_NT, _D)), _row_spec((_NT, _D))),
        out_shape=(jax.ShapeDtypeStruct((_NT, _D), _F32),
                   jax.ShapeDtypeStruct((_NT, _D), _F32)),
    )(x4, t2, p['Wx'], bx, p['Wt'], bt)

    zeros = jnp.zeros((_NT, _D), _F32)

    blocks = p['blocks']
    b0, b1 = blocks[0], blocks[1]

    def blk_weights(b):
        return (b['tt_W1'], b['tt_b1'].reshape(1, _D), b['tt_W2'],
                b['tt_b2'].reshape(1, _D), b['xct_W1'], b['xct_b1'].reshape(1, _D),
                b['xct_W2'], b['xct_b2'].reshape(1, _D), b['out_W'],
                b['out_b'].reshape(1, _D), b['ln_g'].reshape(1, _D),
                b['ln_b'].reshape(1, _D))

    aggt0, aggx = _make_seg_sum((nct, ncx))(
        zeros, tf, et[0], et[1], xs, ex[0], ex[1])

    tf1 = _run_blocked(_blk_body, [tf], [aggt0, aggx], list(blk_weights(b0)),
                       jax.ShapeDtypeStruct((_NT, _D), _F32))

    (aggt1,) = _make_seg_sum((nct,))(zeros, tf1, et[0], et[1])

    mask = unique_mask.reshape(_NT, 1).astype(jnp.int32)
    wsc = p['Wo']
    bsc = p['bo'].reshape(1, 1)
    scores = _run_blocked(
        _final_body, [tf1, mask], [aggt1, aggx],
        list(blk_weights(b1)) + [wsc, bsc],
        jax.ShapeDtypeStruct((_NT, 1), _F32))

    return scores.reshape(nb, nr)


# speedup vs baseline: 14.3739x; 1.4738x over previous
import functools

import jax
import jax.numpy as jnp
from jax import lax
from jax.experimental import pallas as pl
from jax.experimental.pallas import tpu as pltpu
from jax.experimental.pallas import tpu_sc as plsc

_NT = 4096
_D = 128
_CH = 128
_NW = 32
_F32 = jnp.float32


def _seg_sum_body(chunk_counts, *refs):
    n_sets = len(chunk_counts)
    i = 0
    zeros = refs[i]; i += 1
    tables, srcs, dsts = [], [], []
    for _ in range(n_sets):
        tables.append(refs[i]); srcs.append(refs[i + 1]); dsts.append(refs[i + 2])
        i += 3
    outs = refs[i:i + n_sets]; i += n_sets
    scr = refs[i:]
    idx_v = [(scr[2 * k], scr[2 * k + 1]) for k in range(n_sets)]
    rows0 = scr[2 * n_sets]
    rows1 = scr[2 * n_sets + 1]
    shs = scr[2 * n_sets + 2: 2 * n_sets + 2 + n_sets]
    gs0, gs1, ss0, ss1 = scr[-4:]

    c = lax.axis_index("c")
    s = lax.axis_index("s")
    w = c * 16 + s
    rpt = _NT // 16

    for k in range(n_sets):
        pltpu.sync_copy(zeros.at[pl.ds(s * rpt, rpt)], shs[k].at[pl.ds(s * rpt, rpt)])
    plsc.subcore_barrier()

    for k in range(n_sets):
        nper = chunk_counts[k] // _NW
        npairs = nper // 2
        src_v, dst_v = idx_v[k]
        sh = shs[k]
        table = tables[k]
        pltpu.sync_copy(srcs[k].at[pl.ds(w * nper, nper)], src_v)
        pltpu.sync_copy(dsts[k].at[pl.ds(w * nper, nper)], dst_v)

        pltpu.async_copy(table.at[src_v.at[0]], rows0, gs0)
        pltpu.async_copy(table.at[src_v.at[1]], rows1, gs1)

        def loop_body(j2, carry, table=table, src_v=src_v, dst_v=dst_v,
                      sh=sh, npairs=npairs):
            base = 2 * j2
            dummy = table.at[pl.ds(0, _CH)]
            pltpu.make_async_copy(dummy, rows0, gs0).wait()
            pltpu.async_copy(rows0, sh.at[dst_v.at[base]], ss0, add=True)
            pltpu.make_async_copy(dummy, rows0, ss0).wait()

            @pl.when(j2 + 1 < npairs)
            def _():
                pltpu.async_copy(table.at[src_v.at[base + 2]], rows0, gs0)

            pltpu.make_async_copy(dummy, rows1, gs1).wait()
            pltpu.async_copy(rows1, sh.at[dst_v.at[base + 1]], ss1, add=True)
            pltpu.make_async_copy(dummy, rows1, ss1).wait()

            @pl.when(j2 + 1 < npairs)
            def _():
                pltpu.async_copy(table.at[src_v.at[base + 3]], rows1, gs1)

            return carry

        lax.fori_loop(0, npairs, loop_body, 0)

    plsc.subcore_barrier()
    for k in range(n_sets):
        pltpu.sync_copy(shs[k].at[pl.ds(s * rpt, rpt)],
                        outs[k].at[c, pl.ds(s * rpt, rpt)])


@functools.cache
def _make_seg_sum(chunk_counts):
    mesh = plsc.VectorSubcoreMesh(core_axis_name="c", subcore_axis_name="s")
    n_sets = len(chunk_counts)
    out_type = tuple(jax.ShapeDtypeStruct((2, _NT, _D), _F32) for _ in range(n_sets))
    scratch = []
    for nc in chunk_counts:
        nper = nc // _NW
        scratch += [pltpu.VMEM((nper, _CH), jnp.int32),
                    pltpu.VMEM((nper, _CH), jnp.int32)]
    scratch.append(pltpu.VMEM((_CH, _D), _F32))
    scratch.append(pltpu.VMEM((_CH, _D), _F32))
    scratch += [pltpu.VMEM_SHARED((_NT, _D), _F32) for _ in range(n_sets)]
    scratch += [pltpu.SemaphoreType.DMA] * 4
    body = functools.partial(_seg_sum_body, chunk_counts)
    return pl.kernel(body, out_type=out_type, mesh=mesh,
                     scratch_types=tuple(scratch),
                     name=f"seg_sum_{n_sets}")


def _dot(a, b):
    return jnp.dot(a, b, preferred_element_type=_F32)


def _proj_body(x_ref, t_ref, wx_ref, bx_ref, wt_ref, bt_ref, xs_ref, tf_ref):
    xs_ref[...] = jnp.maximum(_dot(x_ref[...], wx_ref[...]) + bx_ref[...], 0.0)
    tf_ref[...] = jnp.maximum(_dot(t_ref[...], wt_ref[...]) + bt_ref[...], 0.0)


def _gin_pair(tf, at0, at1, ax0, ax1, w1t, b1t, w2t, b2t, w1x, b1x, w2x, b2x):
    ht = tf + at0 + at1
    hx = tf + ax0 + ax1
    tt = _dot(jnp.maximum(_dot(ht, w1t) + b1t, 0.0), w2t) + b2t
    xct = _dot(jnp.maximum(_dot(hx, w1x) + b1x, 0.0), w2x) + b2x
    return jnp.maximum(tf + tt + xct, 0.0)


def _post(tf2, wo, bo, g, bl):
    h = jnp.maximum(_dot(tf2, wo) + bo, 0.0)
    mu = jnp.mean(h, axis=-1, keepdims=True)
    var = jnp.mean((h - mu) ** 2, axis=-1, keepdims=True)
    hn = (h - mu) / jnp.sqrt(var + 1e-5) * g + bl
    return tf2 + hn


def _blk_body(tf_ref, at_ref, ax_ref, w1t, b1t, w2t, b2t, w1x, b1x, w2x, b2x,
              wo, bo, g, bl, out_ref):
    tf2 = _gin_pair(tf_ref[...], at_ref[0], at_ref[1], ax_ref[0], ax_ref[1],
                    w1t[...], b1t[...], w2t[...], b2t[...],
                    w1x[...], b1x[...], w2x[...], b2x[...])
    out_ref[...] = _post(tf2, wo[...], bo[...], g[...], bl[...])


def _final_body(tf_ref, mask_ref, at_ref, ax_ref, w1t, b1t, w2t, b2t,
                w1x, b1x, w2x, b2x, wo, bo, g, bl, wsc, bsc, out_ref):
    tf2 = _gin_pair(tf_ref[...], at_ref[0], at_ref[1], ax_ref[0], ax_ref[1],
                    w1t[...], b1t[...], w2t[...], b2t[...],
                    w1x[...], b1x[...], w2x[...], b2x[...])
    tff = _post(tf2, wo[...], bo[...], g[...], bl[...])
    s = _dot(tff, wsc[...]) + bsc[...]
    out_ref[...] = jnp.where(mask_ref[...] != 0, s, -jnp.inf)


_ROWS_BLK = 1024


def _row_spec(shape):
    nd = len(shape)
    blk = (_ROWS_BLK,) + shape[1:]
    if nd == 2:
        return pl.BlockSpec(blk, lambda i: (i, 0))
    return pl.BlockSpec(blk, lambda i: (i, 0, 0))


def _full_spec(shape):
    nd = len(shape)
    return pl.BlockSpec(shape, (lambda i: (0, 0)) if nd == 2 else (lambda i: (0, 0, 0)))


def _part_spec(shape):
    return pl.BlockSpec((2, _ROWS_BLK, shape[2]), lambda i: (0, i, 0))


def _run_blocked(body, row_args, part_args, full_args, out_shape):
    grid = (_NT // _ROWS_BLK,)
    in_specs = ([_row_spec(a.shape) for a in row_args]
                + [_part_spec(a.shape) for a in part_args]
                + [_full_spec(a.shape) for a in full_args])
    return pl.pallas_call(
        body,
        grid=grid,
        in_specs=in_specs,
        out_specs=_row_spec(out_shape.shape),
        out_shape=out_shape,
    )(*row_args, *part_args, *full_args)


def kernel(x, t, e_t, e_xct, unique_mask, params):
    nb, nn_, dn = x.shape
    _, nr, dr = t.shape
    x4 = x.reshape(-1, dn)[:_NT]
    t2 = t.reshape(-1, dr)

    et = e_t.astype(jnp.int32).reshape(2, -1, _CH)
    ex = e_xct.astype(jnp.int32).reshape(2, -1, _CH)
    nct = et.shape[1]
    ncx = ex.shape[1]

    p = params
    bx = p['bx'].reshape(1, _D)
    bt = p['bt'].reshape(1, _D)

    xs, tf = pl.pallas_call(
        _proj_body,
        grid=(_NT // _ROWS_BLK,),
        in_specs=[_row_spec((_NT, _D)), _row_spec((_NT, _D)),
                  _full_spec((_D, _D)), _full_spec((1, _D)),
                  _full_spec((_D, _D)), _full_spec((1, _D))],
        out_specs=(_row_spec((_NT, _D)), _row_spec((_NT, _D))),
        out_shape=(jax.ShapeDtypeStruct((_NT, _D), _F32),
                   jax.ShapeDtypeStruct((_NT, _D), _F32)),
    )(x4, t2, p['Wx'], bx, p['Wt'], bt)

    zeros = jnp.zeros((_NT, _D), _F32)

    blocks = p['blocks']
    b0, b1 = blocks[0], blocks[1]

    def blk_weights(b):
        return (b['tt_W1'], b['tt_b1'].reshape(1, _D), b['tt_W2'],
                b['tt_b2'].reshape(1, _D), b['xct_W1'], b['xct_b1'].reshape(1, _D),
                b['xct_W2'], b['xct_b2'].reshape(1, _D), b['out_W'],
                b['out_b'].reshape(1, _D), b['ln_g'].reshape(1, _D),
                b['ln_b'].reshape(1, _D))

    aggt0, aggx = _make_seg_sum((nct, ncx))(
        zeros, tf, et[0], et[1], xs, ex[0], ex[1])

    tf1 = _run_blocked(_blk_body, [tf], [aggt0, aggx], list(blk_weights(b0)),
                       jax.ShapeDtypeStruct((_NT, _D), _F32))

    (aggt1,) = _make_seg_sum((nct,))(zeros, tf1, et[0], et[1])

    mask = unique_mask.reshape(_NT, 1).astype(jnp.int32)
    wsc = p['Wo']
    bsc = p['bo'].reshape(1, 1)
    scores = _run_blocked(
        _final_body, [tf1, mask], [aggt1, aggx],
        list(blk_weights(b1)) + [wsc, bsc],
        jax.ShapeDtypeStruct((_NT, 1), _F32))

    return scores.reshape(nb, nr)
